# baseline (speedup 1.0000x reference)
import functools

import jax
import jax.numpy as jnp
from jax import lax
from jax.experimental import pallas as pl
from jax.experimental.pallas import tpu as pltpu
from jax.experimental.pallas import tpu_sc as plsc

_T = 128
_N = 32
_C = 3
_H = 224
_W = 224


def _src_frame(w):
    bump = lambda k: jnp.where(w >= k, jnp.int32(1), jnp.int32(0))
    return jnp.int32(4) * w + bump(6) + bump(16) + bump(26)


_CHUNK_ROWS = 56
_K = _H // _CHUNK_ROWS
_NCH = _C * _K
_NB = 8


def _sc_subsample(x):
    mesh = plsc.VectorSubcoreMesh(core_axis_name="c", subcore_axis_name="s")

    @functools.partial(
        pl.kernel,
        mesh=mesh,
        out_type=jax.ShapeDtypeStruct((_N, _C, _H, _W), jnp.float32),
        scratch_types=(
            [pltpu.VMEM((_CHUNK_ROWS, _W), jnp.float32)] * _NB
            + [pltpu.SemaphoreType.DMA] * (2 * _NB)
        ),
    )
    def body(x_hbm, out_hbm, *scratch):
        bufs = scratch[:_NB]
        gsems = scratch[_NB : 2 * _NB]
        ssems = scratch[2 * _NB :]
        w = lax.axis_index("s") * 2 + lax.axis_index("c")
        f = _src_frame(w)

        def src(i):
            c, r = divmod(i, _K)
            return x_hbm.at[f, c, pl.ds(r * _CHUNK_ROWS, _CHUNK_ROWS)]

        def dst(i):
            c, r = divmod(i, _K)
            return out_hbm.at[w, c, pl.ds(r * _CHUNK_ROWS, _CHUNK_ROWS)]

        gath = [None] * _NCH
        scat = [None] * _NCH
        for i in range(min(_NB, _NCH)):
            gath[i] = pltpu.async_copy(src(i), bufs[i], gsems[i])
        for i in range(_NCH):
            b = i % _NB
            gath[i].wait()
            scat[i] = pltpu.async_copy(bufs[b], dst(i), ssems[b])
            j = i + _NB
            if j < _NCH:
                scat[i].wait()
                gath[j] = pltpu.async_copy(src(j), bufs[b], gsems[b])
        for i in range(max(0, _NCH - _NB), _NCH):
            scat[i].wait()

    return body(x)


_IDX = [4 * w + (w >= 6) + (w >= 16) + (w >= 26) for w in range(_N)]
_NQ = 8


def _tc_subsample(x):
    def body(x_ref, o_ref, vbuf, gsem, ssem):
        gcps = [
            pltpu.make_async_copy(x_ref.at[_IDX[w]], vbuf.at[w], gsem.at[w])
            for w in range(_N)
        ]
        for cp in gcps:
            cp.start()
        scps = []
        for w in range(_N):
            gcps[w].wait()
            scp = pltpu.make_async_copy(vbuf.at[w], o_ref.at[w], ssem.at[w])
            scp.start()
            scps.append(scp)
        for cp in scps:
            cp.wait()

    return pl.pallas_call(
        body,
        in_specs=[pl.BlockSpec(memory_space=pltpu.MemorySpace.HBM)],
        out_specs=pl.BlockSpec(memory_space=pltpu.MemorySpace.HBM),
        out_shape=jax.ShapeDtypeStruct((_N, _C, _H, _W), jnp.float32),
        scratch_shapes=[
            pltpu.VMEM((_N, _C, _H, _W), jnp.float32),
            pltpu.SemaphoreType.DMA((_N,)),
            pltpu.SemaphoreType.DMA((_N,)),
        ],
    )(x)


_BH = 56


def _tc_lane_gather(x):
    xt = jnp.transpose(x, (1, 2, 3, 0))

    def body(x_ref, o_ref):
        w = jax.lax.iota(jnp.int32, _N)
        idx = _src_frame(w)
        a = x_ref[...].reshape(_BH * _W, _T)
        ib = jnp.broadcast_to(idx[None, :], (_BH * _W, _N))
        o_ref[...] = jnp.take_along_axis(a, ib, axis=1).reshape(
            1, _BH, _W, _N
        )

    y = pl.pallas_call(
        body,
        grid=(_C, _H // _BH),
        in_specs=[
            pl.BlockSpec((1, _BH, _W, _T), lambda c, h: (c, h, 0, 0))
        ],
        out_specs=pl.BlockSpec((1, _BH, _W, _N), lambda c, h: (c, h, 0, 0)),
        out_shape=jax.ShapeDtypeStruct((_C, _H, _W, _N), jnp.float32),
    )(xt)
    return jnp.transpose(y, (3, 0, 1, 2))


def kernel(x):
    return _tc_lane_gather(x)


# speedup vs baseline: 1.1849x; 1.1849x over previous
import functools

import jax
import jax.numpy as jnp
from jax import lax
from jax.experimental import pallas as pl
from jax.experimental.pallas import tpu as pltpu
from jax.experimental.pallas import tpu_sc as plsc

_T = 128
_N = 32
_C = 3
_H = 224
_W = 224


def _src_frame(w):
    bump = lambda k: jnp.where(w >= k, jnp.int32(1), jnp.int32(0))
    return jnp.int32(4) * w + bump(6) + bump(16) + bump(26)


_CHUNK_ROWS = 56
_K = _H // _CHUNK_ROWS
_NCH = _C * _K
_NB = 8


def _sc_subsample(x):
    mesh = plsc.VectorSubcoreMesh(core_axis_name="c", subcore_axis_name="s")

    @functools.partial(
        pl.kernel,
        mesh=mesh,
        out_type=jax.ShapeDtypeStruct((_N, _C, _H, _W), jnp.float32),
        scratch_types=(
            [pltpu.VMEM((_CHUNK_ROWS, _W), jnp.float32)] * _NB
            + [pltpu.SemaphoreType.DMA] * (2 * _NB)
        ),
    )
    def body(x_hbm, out_hbm, *scratch):
        bufs = scratch[:_NB]
        gsems = scratch[_NB : 2 * _NB]
        ssems = scratch[2 * _NB :]
        w = lax.axis_index("s") * 2 + lax.axis_index("c")
        f = _src_frame(w)

        def src(i):
            c, r = divmod(i, _K)
            return x_hbm.at[f, c, pl.ds(r * _CHUNK_ROWS, _CHUNK_ROWS)]

        def dst(i):
            c, r = divmod(i, _K)
            return out_hbm.at[w, c, pl.ds(r * _CHUNK_ROWS, _CHUNK_ROWS)]

        gath = [None] * _NCH
        scat = [None] * _NCH
        for i in range(min(_NB, _NCH)):
            gath[i] = pltpu.async_copy(src(i), bufs[i], gsems[i])
        for i in range(_NCH):
            b = i % _NB
            gath[i].wait()
            scat[i] = pltpu.async_copy(bufs[b], dst(i), ssems[b])
            j = i + _NB
            if j < _NCH:
                scat[i].wait()
                gath[j] = pltpu.async_copy(src(j), bufs[b], gsems[b])
        for i in range(max(0, _NCH - _NB), _NCH):
            scat[i].wait()

    return body(x)


_IDX = [4 * w + (w >= 6) + (w >= 16) + (w >= 26) for w in range(_N)]
_NQ = 8


def _tc_subsample(x):
    def body(x_ref, o_ref, vbuf, gsem, ssem):
        gcps = [
            pltpu.make_async_copy(x_ref.at[_IDX[w]], vbuf.at[w], gsem.at[w])
            for w in range(_N)
        ]
        for cp in gcps:
            cp.start()
        scps = []
        for w in range(_N):
            gcps[w].wait()
            scp = pltpu.make_async_copy(vbuf.at[w], o_ref.at[w], ssem.at[w])
            scp.start()
            scps.append(scp)
        for cp in scps:
            cp.wait()

    return pl.pallas_call(
        body,
        in_specs=[pl.BlockSpec(memory_space=pltpu.MemorySpace.HBM)],
        out_specs=pl.BlockSpec(memory_space=pltpu.MemorySpace.HBM),
        out_shape=jax.ShapeDtypeStruct((_N, _C, _H, _W), jnp.float32),
        scratch_shapes=[
            pltpu.VMEM((_N, _C, _H, _W), jnp.float32),
            pltpu.SemaphoreType.DMA((_N,)),
            pltpu.SemaphoreType.DMA((_N,)),
        ],
    )(x)


_BH = 56


def _tc_lane_gather(x):
    xt = jnp.transpose(x, (1, 2, 3, 0))

    def body(x_ref, o_ref):
        w = jax.lax.iota(jnp.int32, _N)
        idx = _src_frame(w)
        sel = (idx[:, None] == jax.lax.iota(jnp.int32, _T)[None, :]).astype(
            jnp.float32
        )
        a = x_ref[...].reshape(_BH * _W, _T)
        ob = jax.lax.dot_general(
            sel,
            a,
            (((1,), (1,)), ((), ())),
            precision=jax.lax.Precision.HIGHEST,
        )
        o_ref[...] = ob.reshape(_N, 1, _BH, _W)

    return pl.pallas_call(
        body,
        grid=(_C, _H // _BH),
        in_specs=[
            pl.BlockSpec((1, _BH, _W, _T), lambda c, h: (c, h, 0, 0))
        ],
        out_specs=pl.BlockSpec((_N, 1, _BH, _W), lambda c, h: (0, c, h, 0)),
        out_shape=jax.ShapeDtypeStruct((_N, _C, _H, _W), jnp.float32),
    )(xt)


def kernel(x):
    return _tc_lane_gather(x)


# speedup vs baseline: 2.5221x; 2.1285x over previous
import functools

import jax
import jax.numpy as jnp
from jax import lax
from jax.experimental import pallas as pl
from jax.experimental.pallas import tpu as pltpu
from jax.experimental.pallas import tpu_sc as plsc

_T = 128
_N = 32
_C = 3
_H = 224
_W = 224


def _src_frame(w):
    bump = lambda k: jnp.where(w >= k, jnp.int32(1), jnp.int32(0))
    return jnp.int32(4) * w + bump(6) + bump(16) + bump(26)


_CHUNK_ROWS = 56
_K = _H // _CHUNK_ROWS
_NCH = _C * _K
_NB = 8


def _sc_subsample(x):
    mesh = plsc.VectorSubcoreMesh(core_axis_name="c", subcore_axis_name="s")

    @functools.partial(
        pl.kernel,
        mesh=mesh,
        out_type=jax.ShapeDtypeStruct((_N, _C, _H, _W), jnp.float32),
        scratch_types=(
            [pltpu.VMEM((_CHUNK_ROWS, _W), jnp.float32)] * _NB
            + [pltpu.SemaphoreType.DMA] * (2 * _NB)
        ),
    )
    def body(x_hbm, out_hbm, *scratch):
        bufs = scratch[:_NB]
        gsems = scratch[_NB : 2 * _NB]
        ssems = scratch[2 * _NB :]
        w = lax.axis_index("s") * 2 + lax.axis_index("c")
        f = _src_frame(w)

        def src(i):
            c, r = divmod(i, _K)
            return x_hbm.at[f, c, pl.ds(r * _CHUNK_ROWS, _CHUNK_ROWS)]

        def dst(i):
            c, r = divmod(i, _K)
            return out_hbm.at[w, c, pl.ds(r * _CHUNK_ROWS, _CHUNK_ROWS)]

        gath = [None] * _NCH
        scat = [None] * _NCH
        for i in range(min(_NB, _NCH)):
            gath[i] = pltpu.async_copy(src(i), bufs[i], gsems[i])
        for i in range(_NCH):
            b = i % _NB
            gath[i].wait()
            scat[i] = pltpu.async_copy(bufs[b], dst(i), ssems[b])
            j = i + _NB
            if j < _NCH:
                scat[i].wait()
                gath[j] = pltpu.async_copy(src(j), bufs[b], gsems[b])
        for i in range(max(0, _NCH - _NB), _NCH):
            scat[i].wait()

    return body(x)


_IDX = [4 * w + (w >= 6) + (w >= 16) + (w >= 26) for w in range(_N)]
_NQ = 8


def _tc_subsample(x):
    def body(x_ref, o_ref, vbuf, gsem, ssem):
        gcps = [
            pltpu.make_async_copy(x_ref.at[_IDX[w]], vbuf.at[w], gsem.at[w])
            for w in range(_N)
        ]
        for cp in gcps:
            cp.start()
        scps = []
        for w in range(_N):
            gcps[w].wait()
            scp = pltpu.make_async_copy(vbuf.at[w], o_ref.at[w], ssem.at[w])
            scp.start()
            scps.append(scp)
        for cp in scps:
            cp.wait()

    return pl.pallas_call(
        body,
        in_specs=[pl.BlockSpec(memory_space=pltpu.MemorySpace.HBM)],
        out_specs=pl.BlockSpec(memory_space=pltpu.MemorySpace.HBM),
        out_shape=jax.ShapeDtypeStruct((_N, _C, _H, _W), jnp.float32),
        scratch_shapes=[
            pltpu.VMEM((_N, _C, _H, _W), jnp.float32),
            pltpu.SemaphoreType.DMA((_N,)),
            pltpu.SemaphoreType.DMA((_N,)),
        ],
    )(x)


_BH = 56


def _tc_lane_gather(x):
    xt = jnp.transpose(x, (1, 2, 3, 0))

    def body(x_ref, o_ref):
        a = x_ref[...].reshape(_BH * _W, _T)
        at = jnp.transpose(a)
        ob = jnp.concatenate([at[i : i + 1] for i in _IDX], axis=0)
        o_ref[...] = ob.reshape(_N, 1, _BH, _W)

    return pl.pallas_call(
        body,
        grid=(_C, _H // _BH),
        in_specs=[
            pl.BlockSpec((1, _BH, _W, _T), lambda c, h: (c, h, 0, 0))
        ],
        out_specs=pl.BlockSpec((_N, 1, _BH, _W), lambda c, h: (0, c, h, 0)),
        out_shape=jax.ShapeDtypeStruct((_N, _C, _H, _W), jnp.float32),
    )(xt)


def kernel(x):
    return _tc_lane_gather(x)


# speedup vs baseline: 2.5997x; 1.0308x over previous
import functools

import jax
import jax.numpy as jnp
from jax import lax
from jax.experimental import pallas as pl
from jax.experimental.pallas import tpu as pltpu
from jax.experimental.pallas import tpu_sc as plsc

_T = 128
_N = 32
_C = 3
_H = 224
_W = 224


def _src_frame(w):
    bump = lambda k: jnp.where(w >= k, jnp.int32(1), jnp.int32(0))
    return jnp.int32(4) * w + bump(6) + bump(16) + bump(26)


_CHUNK_ROWS = 56
_K = _H // _CHUNK_ROWS
_NCH = _C * _K
_NB = 8


def _sc_subsample(x):
    mesh = plsc.VectorSubcoreMesh(core_axis_name="c", subcore_axis_name="s")

    @functools.partial(
        pl.kernel,
        mesh=mesh,
        out_type=jax.ShapeDtypeStruct((_N, _C, _H, _W), jnp.float32),
        scratch_types=(
            [pltpu.VMEM((_CHUNK_ROWS, _W), jnp.float32)] * _NB
            + [pltpu.SemaphoreType.DMA] * (2 * _NB)
        ),
    )
    def body(x_hbm, out_hbm, *scratch):
        bufs = scratch[:_NB]
        gsems = scratch[_NB : 2 * _NB]
        ssems = scratch[2 * _NB :]
        w = lax.axis_index("s") * 2 + lax.axis_index("c")
        f = _src_frame(w)

        def src(i):
            c, r = divmod(i, _K)
            return x_hbm.at[f, c, pl.ds(r * _CHUNK_ROWS, _CHUNK_ROWS)]

        def dst(i):
            c, r = divmod(i, _K)
            return out_hbm.at[w, c, pl.ds(r * _CHUNK_ROWS, _CHUNK_ROWS)]

        gath = [None] * _NCH
        scat = [None] * _NCH
        for i in range(min(_NB, _NCH)):
            gath[i] = pltpu.async_copy(src(i), bufs[i], gsems[i])
        for i in range(_NCH):
            b = i % _NB
            gath[i].wait()
            scat[i] = pltpu.async_copy(bufs[b], dst(i), ssems[b])
            j = i + _NB
            if j < _NCH:
                scat[i].wait()
                gath[j] = pltpu.async_copy(src(j), bufs[b], gsems[b])
        for i in range(max(0, _NCH - _NB), _NCH):
            scat[i].wait()

    return body(x)


_IDX = [4 * w + (w >= 6) + (w >= 16) + (w >= 26) for w in range(_N)]
_NQ = 8


def _tc_subsample(x):
    def body(x_ref, o_ref, vbuf, gsem, ssem):
        gcps = [
            pltpu.make_async_copy(x_ref.at[_IDX[w]], vbuf.at[w], gsem.at[w])
            for w in range(_N)
        ]
        for cp in gcps:
            cp.start()
        scps = []
        for w in range(_N):
            gcps[w].wait()
            scp = pltpu.make_async_copy(vbuf.at[w], o_ref.at[w], ssem.at[w])
            scp.start()
            scps.append(scp)
        for cp in scps:
            cp.wait()

    return pl.pallas_call(
        body,
        in_specs=[pl.BlockSpec(memory_space=pltpu.MemorySpace.HBM)],
        out_specs=pl.BlockSpec(memory_space=pltpu.MemorySpace.HBM),
        out_shape=jax.ShapeDtypeStruct((_N, _C, _H, _W), jnp.float32),
        scratch_shapes=[
            pltpu.VMEM((_N, _C, _H, _W), jnp.float32),
            pltpu.SemaphoreType.DMA((_N,)),
            pltpu.SemaphoreType.DMA((_N,)),
        ],
    )(x)


_BH = 112


def _tc_lane_gather(x):
    xt = jnp.transpose(x, (1, 2, 3, 0))

    def body(x_ref, o_ref):
        a = x_ref[...].reshape(_BH * _W, _T)
        at = jnp.transpose(a)
        ob = jnp.concatenate([at[i : i + 1] for i in _IDX], axis=0)
        o_ref[...] = ob.reshape(_N, 1, _BH, _W)

    return pl.pallas_call(
        body,
        grid=(_C, _H // _BH),
        in_specs=[
            pl.BlockSpec((1, _BH, _W, _T), lambda c, h: (c, h, 0, 0))
        ],
        out_specs=pl.BlockSpec((_N, 1, _BH, _W), lambda c, h: (0, c, h, 0)),
        out_shape=jax.ShapeDtypeStruct((_N, _C, _H, _W), jnp.float32),
    )(xt)


def kernel(x):
    return _tc_lane_gather(x)


# speedup vs baseline: 2.6116x; 1.0046x over previous
import jax
import jax.numpy as jnp
from jax.experimental import pallas as pl

_T = 128
_N = 32
_C = 3
_H = 224
_W = 224

_IDX = [4 * w + (w >= 6) + (w >= 16) + (w >= 26) for w in range(_N)]

_BH = 112


def _tc_lane_gather(x):
    xt = jnp.transpose(x, (1, 2, 3, 0))

    def body(x_ref, o_ref):
        a = x_ref[...].reshape(_BH * _W, _T)
        at = jnp.transpose(a)
        ob = jnp.concatenate([at[i : i + 1] for i in _IDX], axis=0)
        o_ref[...] = ob.reshape(_N, 1, _BH, _W)

    return pl.pallas_call(
        body,
        grid=(_C, _H // _BH),
        in_specs=[
            pl.BlockSpec((1, _BH, _W, _T), lambda c, h: (c, h, 0, 0))
        ],
        out_specs=pl.BlockSpec((_N, 1, _BH, _W), lambda c, h: (0, c, h, 0)),
        out_shape=jax.ShapeDtypeStruct((_N, _C, _H, _W), jnp.float32),
    )(xt)


def kernel(x):
    return _tc_lane_gather(x)
